# baseline (speedup 1.0000x reference)
import functools

import jax
import jax.numpy as jnp
from jax import lax
from jax.experimental import pallas as pl
from jax.experimental.pallas import tpu as pltpu
from jax.experimental.pallas import tpu_sc as plsc

BATCH = 16384
NUM_CORES = 2
NUM_SUBCORES = 16
LANES = 16
NUM_WORKERS = NUM_CORES * NUM_SUBCORES
B_PER_W = BATCH // NUM_WORKERS
NUM_ROWS = 1000000
HALF = NUM_ROWS // 2
QTR = NUM_ROWS // 4


def _bias_body(uid_hbm, iid_hbm,
               utabA_hbm, utabB_hbm, utabC_hbm, utabD_hbm,
               itabA_hbm, itabB_hbm, itabC_hbm, itabD_hbm,
               out_hbm, uidx_v, iidx_v, umod_v, imod_v,
               uvalA_v, uvalB_v, uvalC_v, uvalD_v,
               ivalA_v, ivalB_v, ivalC_v, ivalD_v, acc_v, sem):
    wid = lax.axis_index("s") * NUM_CORES + lax.axis_index("c")
    base = wid * B_PER_W

    cp_u = pltpu.async_copy(uid_hbm.at[pl.ds(base, B_PER_W)], uidx_v, sem)
    cp_i = pltpu.async_copy(iid_hbm.at[pl.ds(base, B_PER_W)], iidx_v, sem)
    cp_u.wait()
    cp_i.wait()

    @pl.loop(0, B_PER_W, step=LANES)
    def _(c):
        s = pl.ds(c, LANES)
        umod_v[s] = lax.rem(uidx_v[s], QTR)
        imod_v[s] = lax.rem(iidx_v[s], QTR)

    gathers = [
        pltpu.async_copy(tab.at[idx], val, sem)
        for tab, idx, val in (
            (utabA_hbm, umod_v, uvalA_v),
            (utabB_hbm, umod_v, uvalB_v),
            (utabC_hbm, umod_v, uvalC_v),
            (utabD_hbm, umod_v, uvalD_v),
            (itabA_hbm, imod_v, ivalA_v),
            (itabB_hbm, imod_v, ivalB_v),
            (itabC_hbm, imod_v, ivalC_v),
            (itabD_hbm, imod_v, ivalD_v),
        )
    ]
    for g in gathers:
        g.wait()

    @pl.loop(0, B_PER_W, step=LANES)
    def _(c):
        s = pl.ds(c, LANES)
        u = uidx_v[s]
        i = iidx_v[s]
        uval = jnp.where(
            u < 2 * QTR,
            jnp.where(u < QTR, uvalA_v[s], uvalB_v[s]),
            jnp.where(u < 3 * QTR, uvalC_v[s], uvalD_v[s]),
        )
        ival = jnp.where(
            i < 2 * QTR,
            jnp.where(i < QTR, ivalA_v[s], ivalB_v[s]),
            jnp.where(i < 3 * QTR, ivalC_v[s], ivalD_v[s]),
        )
        acc_v[s] = uval + ival

    pltpu.sync_copy(acc_v, out_hbm.at[pl.ds(base, B_PER_W)])


def _squeeze_quarters(table):
    return tuple(
        table[i * QTR:(i + 1) * QTR].reshape(-1) for i in range(4)
    )


@jax.jit
def kernel(user_ids, item_ids, user_bias_table, item_bias_table):
    mesh = plsc.VectorSubcoreMesh(core_axis_name="c", subcore_axis_name="s")
    sc_kernel = pl.kernel(
        _bias_body,
        out_type=jax.ShapeDtypeStruct((BATCH,), jnp.float32),
        mesh=mesh,
        scratch_types=(
            [pltpu.VMEM((B_PER_W,), jnp.int32) for _ in range(4)]
            + [pltpu.VMEM((B_PER_W,), jnp.float32) for _ in range(9)]
            + [pltpu.SemaphoreType.DMA]
        ),
    )
    return sc_kernel(
        user_ids.astype(jnp.int32),
        item_ids.astype(jnp.int32),
        *_squeeze_quarters(user_bias_table),
        *_squeeze_quarters(item_bias_table),
    )


# speedup vs baseline: 1.1064x; 1.1064x over previous
import functools

import jax
import jax.numpy as jnp
from jax import lax
from jax.experimental import pallas as pl
from jax.experimental.pallas import tpu as pltpu
from jax.experimental.pallas import tpu_sc as plsc

BATCH = 16384
NUM_CORES = 2
NUM_SUBCORES = 16
LANES = 16
NUM_WORKERS = NUM_CORES * NUM_SUBCORES
B_PER_W = BATCH // NUM_WORKERS
NUM_ROWS = 1000000
HALF = NUM_ROWS // 2


def _bias_body(uid_hbm, iid_hbm, utabA_hbm, utabB_hbm, itabA_hbm, itabB_hbm,
               out_hbm, uidx_v, iidx_v, umod_v, imod_v,
               uvalA_v, uvalB_v, ivalA_v, ivalB_v, acc_v, sem):
    wid = lax.axis_index("s") * NUM_CORES + lax.axis_index("c")
    base = wid * B_PER_W

    cp_u = pltpu.async_copy(uid_hbm.at[pl.ds(base, B_PER_W)], uidx_v, sem)
    cp_i = pltpu.async_copy(iid_hbm.at[pl.ds(base, B_PER_W)], iidx_v, sem)
    cp_u.wait()
    cp_i.wait()

    @pl.loop(0, B_PER_W, step=LANES)
    def _(c):
        s = pl.ds(c, LANES)
        umod_v[s] = lax.rem(uidx_v[s], HALF)
        imod_v[s] = lax.rem(iidx_v[s], HALF)

    g0 = pltpu.async_copy(utabA_hbm.at[umod_v], uvalA_v, sem)
    g1 = pltpu.async_copy(utabB_hbm.at[umod_v], uvalB_v, sem)
    g2 = pltpu.async_copy(itabA_hbm.at[imod_v], ivalA_v, sem)
    g3 = pltpu.async_copy(itabB_hbm.at[imod_v], ivalB_v, sem)
    g0.wait()
    g1.wait()
    g2.wait()
    g3.wait()

    @pl.loop(0, B_PER_W, step=LANES)
    def _(c):
        s = pl.ds(c, LANES)
        u = uidx_v[s]
        i = iidx_v[s]
        uval = jnp.where(u < HALF, uvalA_v[s], uvalB_v[s])
        ival = jnp.where(i < HALF, ivalA_v[s], ivalB_v[s])
        acc_v[s] = uval + ival

    pltpu.sync_copy(acc_v, out_hbm.at[pl.ds(base, B_PER_W)])


def _squeeze_halves(table):
    return (table[:HALF].reshape(-1), table[HALF:].reshape(-1))


@jax.jit
def kernel(user_ids, item_ids, user_bias_table, item_bias_table):
    mesh = plsc.VectorSubcoreMesh(core_axis_name="c", subcore_axis_name="s")
    sc_kernel = pl.kernel(
        _bias_body,
        out_type=jax.ShapeDtypeStruct((BATCH,), jnp.float32),
        mesh=mesh,
        scratch_types=(
            [pltpu.VMEM((B_PER_W,), jnp.int32) for _ in range(4)]
            + [pltpu.VMEM((B_PER_W,), jnp.float32) for _ in range(5)]
            + [pltpu.SemaphoreType.DMA]
        ),
    )
    utabA, utabB = _squeeze_halves(user_bias_table)
    itabA, itabB = _squeeze_halves(item_bias_table)
    return sc_kernel(
        user_ids.astype(jnp.int32),
        item_ids.astype(jnp.int32),
        utabA, utabB, itabA, itabB,
    )
